# Initial kernel scaffold; baseline (speedup 1.0000x reference)
import jax
import jax.numpy as jnp
from jax.experimental import pallas as pl


def kernel(xyz, A1, A2, A3, A4, A5, B1, B2, B3, B4, B5, M1w, M1b, M2w, M2b, Rw, Rb):
    raise NotImplementedError("write your pallas kernel here")


# speedup vs baseline: 11.7679x; 11.7679x over previous
import functools

import jax
import jax.numpy as jnp
from jax import lax
from jax.experimental import pallas as pl

F32 = jnp.float32
NEG = -3.0e38
BIG = 3.0e38


def _mm(a, b):
    return lax.dot_general(a, b, (((1,), (0,)), ((), ())),
                           preferred_element_type=F32,
                           precision=lax.Precision.HIGHEST)


def _mmb(a, b):
    return lax.dot_general(a.astype(jnp.bfloat16), b.astype(jnp.bfloat16),
                           (((1,), (0,)), ((), ())),
                           preferred_element_type=F32)


def _mmTTb(a, b):
    return lax.dot_general(a.astype(jnp.bfloat16), b.astype(jnp.bfloat16),
                           (((0,), (0,)), ((), ())),
                           preferred_element_type=F32)


def _mmT(a, b):
    return lax.dot_general(a, b, (((1,), (1,)), ((), ())),
                           preferred_element_type=F32,
                           precision=lax.Precision.HIGHEST)


def _mmTT(a, b):
    return lax.dot_general(a, b, (((0,), (0,)), ((), ())),
                           preferred_element_type=F32,
                           precision=lax.Precision.HIGHEST)


def _fps_body(xs_ref, out_ref):
    x0 = xs_ref[0]
    x1 = xs_ref[1]
    x2 = xs_ref[2]
    pidx = (lax.broadcasted_iota(jnp.int32, (8, 512), 0) * 512
            + lax.broadcasted_iota(jnp.int32, (8, 512), 1))
    rsel = lax.broadcasted_iota(jnp.int32, (8, 1024), 0)
    csel = lax.broadcasted_iota(jnp.int32, (8, 1024), 1)

    def body(i, st):
        dist, far, ps = st
        fm = pidx == far
        c0 = jnp.sum(jnp.where(fm, x0, 0.0))
        c1 = jnp.sum(jnp.where(fm, x1, 0.0))
        c2 = jnp.sum(jnp.where(fm, x2, 0.0))
        d = (x0 - c0) ** 2 + (x1 - c1) ** 2 + (x2 - c2) ** 2
        dist = jnp.minimum(dist, d)
        m = jnp.max(dist)
        far = jnp.min(jnp.where(dist == m, pidx, jnp.int32(1 << 30)))
        cb = jnp.where(rsel == 0, c0, jnp.where(rsel == 1, c1,
                       jnp.where(rsel == 2, c2, 0.0)))
        ps = jnp.where(csel == i, cb, ps)
        return dist, far, ps

    dist0 = jnp.full((8, 512), 1e10, F32)
    ps0 = jnp.zeros((8, 1024), F32)
    _, _, ps = lax.fori_loop(0, 1024, body, (dist0, jnp.int32(0), ps0))
    out_ref[...] = ps


def _fps(xs3):
    return pl.pallas_call(
        _fps_body,
        out_shape=jax.ShapeDtypeStruct((8, 1024), F32),
    )(xs3)


def _knn_patch_body(q_ref, ref_ref, out_ref, *, k):
    q = q_ref[...]
    r = ref_ref[...]
    n = r.shape[1]
    rn = jnp.sum(r * r, axis=0, keepdims=True)
    inner = _mmTTb(q, r)
    io0 = lax.broadcasted_iota(jnp.int32, (128, 128), 0)
    io1 = lax.broadcasted_iota(jnp.int32, (128, 128), 1)
    eyef = (io0 == io1).astype(F32)
    qn_row = jnp.sum(q * q, axis=0, keepdims=True)
    qn_col = _mmT(eyef, qn_row)
    D = (qn_col - 2.0 * inner) + rn
    iol = lax.broadcasted_iota(jnp.int32, (128, n), 1)
    for s in range(k):
        m = jnp.min(D, axis=1, keepdims=True)
        idx = jnp.min(jnp.where(D == m, iol, jnp.int32(1 << 30)),
                      axis=1, keepdims=True)
        oh = iol == idx
        if s > 0:
            patch = _mmT(r, oh.astype(F32)) - q
            out_ref[s - 1] = patch
        D = jnp.where(oh, BIG, D)


def _knn_patch(ps8, ref8, k):
    n = ref8.shape[1]
    return pl.pallas_call(
        functools.partial(_knn_patch_body, k=k),
        grid=(8,),
        in_specs=[
            pl.BlockSpec((8, 128), lambda b: (0, b)),
            pl.BlockSpec((8, n), lambda b: (0, 0)),
        ],
        out_specs=pl.BlockSpec((k - 1, 8, 128), lambda b: (0, 0, b)),
        out_shape=jax.ShapeDtypeStruct((k - 1, 8, 1024), F32),
    )(ps8, ref8)


def _seg_max4(a):
    n = a.shape[1] // 4
    return jnp.maximum(
        jnp.maximum(a[:, :n], a[:, n:2 * n]),
        jnp.maximum(a[:, 2 * n:3 * n], a[:, 3 * n:]))


def _loc_body(x_ref, w1, w2, w3, w4, w5, out_ref):
    x0 = x_ref[...]
    a = _mmb(w1[:, 0:3], x0[0:3, :])
    bv = _mmb(w1[:, 3:6], x0[0:3, :])
    t = _seg_max4(a)
    x1 = jax.nn.relu(jnp.concatenate([t, t, t, t], axis=1) + bv)
    a = _mmb(w2[:, 0:64], x1)
    bv = _mmb(w2[:, 64:128], x1)
    t = _seg_max4(a)
    x2 = jax.nn.relu(jnp.concatenate([t, t, t, t], axis=1) + bv)
    a = _mmb(w3[:, 0:64], x2)
    bv = _mmb(w3[:, 64:128], x2)
    t = _seg_max4(a)
    x3 = jax.nn.relu(jnp.concatenate([t, t, t, t], axis=1) + bv)
    a = _mmb(w4[:, 0:128], x3)
    bv = _mmb(w4[:, 128:256], x3)
    t = _seg_max4(a)
    x4 = jax.nn.relu(jnp.concatenate([t, t, t, t], axis=1) + bv)
    cat = jnp.concatenate([x1, x2, x3, x4], axis=0)
    h = _mmb(w5[...], cat)
    out_ref[...] = jnp.tanh(_seg_max4(h))


def _loc_dgcnn(x, ws):
    return pl.pallas_call(
        _loc_body,
        out_shape=jax.ShapeDtypeStruct((512, 1024), F32),
    )(x, *ws)


def _se_layer_body(x_ref, w_ref, out_ref, *, cin):
    xb = x_ref[...]
    a = _mmb(w_ref[:, 0:cin], xb[0:cin, :])
    bv = _mmb(w_ref[:, cin:2 * cin], xb[0:cin, :])
    g = _mmTTb(xb, xb)
    io0 = lax.broadcasted_iota(jnp.int32, (128, 128), 0)
    io1 = lax.broadcasted_iota(jnp.int32, (128, 128), 1)
    n_row = jnp.sum(xb * xb, axis=0, keepdims=True)
    eyef = (io0 == io1).astype(F32)
    n_col = _mmT(eyef, n_row)
    pd = (-n_row - (-2.0 * g)) - n_col
    grp = (io0 // 16) == (io1 // 16)
    pdm = jnp.where(grp, pd, NEG)
    m_acc = jnp.full(a.shape, NEG, F32)
    for _ in range(4):
        mrow = jnp.max(pdm, axis=1, keepdims=True)
        idx = jnp.min(jnp.where(pdm == mrow, io1, jnp.int32(1 << 30)),
                      axis=1, keepdims=True)
        oh = io1 == idx
        m_acc = jnp.maximum(m_acc, _mmT(a, oh.astype(F32)))
        pdm = jnp.where(oh, NEG, pdm)
    out_ref[...] = jax.nn.relu(m_acc + bv)


def _se_layer(x, w, cin, cout):
    return pl.pallas_call(
        functools.partial(_se_layer_body, cin=cin),
        grid=(128,),
        in_specs=[
            pl.BlockSpec((x.shape[0], 128), lambda b: (0, b)),
            pl.BlockSpec(w.shape, lambda b: (0, 0)),
        ],
        out_specs=pl.BlockSpec((cout, 128), lambda b: (0, b)),
        out_shape=jax.ShapeDtypeStruct((cout, 16384), F32),
    )(x, w)


def _head_body(cat_ref, a5_ref, loc_ref, m1w_ref, m1b_ref, m2w_ref, m2b_ref,
               rw_ref, rb_ref, out_ref):
    a5 = a5_ref[...]
    m = jnp.full((512, 1024), NEG, F32)
    for p in range(16):
        m = jnp.maximum(m, _mmb(a5, cat_ref[:, p * 1024:(p + 1) * 1024]))
    se = jnp.tanh(m)
    loc = loc_ref[...]
    h = jax.nn.relu(_mmb(m1w_ref[:, 0:512], se)
                    + _mmb(m1w_ref[:, 512:1024], loc) + m1b_ref[...])
    h = jax.nn.relu(_mmb(m2w_ref[...], h) + m2b_ref[...])
    f = _mmb(rw_ref[...], h) + rb_ref[...]
    out_ref[...] = jnp.concatenate([f, jnp.zeros((5, 1024), F32)], axis=0)


def _head(cat_p, a5, loc, m1w, m1b, m2w, m2b, rw, rb):
    return pl.pallas_call(
        _head_body,
        out_shape=jax.ShapeDtypeStruct((8, 1024), F32),
    )(cat_p, a5, loc, m1w, m1b, m2w, m2b, rw, rb)


def kernel(xyz, A1, A2, A3, A4, A5, B1, B2, B3, B4, B5,
           M1w, M1b, M2w, M2b, Rw, Rb):
    xs = xyz[0]
    xs3 = xs.reshape(3, 8, 512)
    xs8 = jnp.concatenate([xs, jnp.zeros((5, 4096), F32)], axis=0)

    ps8 = _fps(xs3)

    p1 = _knn_patch(ps8, xs8, 5)
    xloc = p1.transpose(1, 0, 2).reshape(8, 4096)
    loc = _loc_dgcnn(xloc, (B1, B2, B3, B4, B5))

    p2 = _knn_patch(ps8, ps8, 17)
    xse = p2.transpose(1, 2, 0).reshape(8, 16384)
    x1 = _se_layer(xse, A1, 3, 64)
    x2 = _se_layer(x1, A2, 64, 64)
    x3 = _se_layer(x2, A3, 64, 128)
    x4 = _se_layer(x3, A4, 128, 256)
    cat = jnp.concatenate([x1, x2, x3, x4], axis=0)
    cat_p = cat.reshape(512, 1024, 16).transpose(0, 2, 1).reshape(512, 16384)

    feat8 = _head(cat_p, A5, loc, M1w, M1b.reshape(512, 1),
                  M2w, M2b.reshape(256, 1), Rw, Rb.reshape(3, 1))
    ps_out = ps8[0:3][None]
    feat = feat8[0:3][None]
    return (ps_out, feat)


# speedup vs baseline: 15.6425x; 1.3292x over previous
import functools

import jax
import jax.numpy as jnp
from jax import lax
from jax.experimental import pallas as pl
from jax.experimental.pallas import tpu as pltpu

F32 = jnp.float32
NEG = -3.0e38
BIG = 3.0e38


def _mm(a, b):
    return lax.dot_general(a, b, (((1,), (0,)), ((), ())),
                           preferred_element_type=F32,
                           precision=lax.Precision.HIGHEST)


def _mmb(a, b):
    return lax.dot_general(a.astype(jnp.bfloat16), b.astype(jnp.bfloat16),
                           (((1,), (0,)), ((), ())),
                           preferred_element_type=F32)


def _mmTTb(a, b):
    return lax.dot_general(a.astype(jnp.bfloat16), b.astype(jnp.bfloat16),
                           (((0,), (0,)), ((), ())),
                           preferred_element_type=F32)


def _mmT(a, b):
    return lax.dot_general(a, b, (((1,), (1,)), ((), ())),
                           preferred_element_type=F32,
                           precision=lax.Precision.HIGHEST)


def _mmTb(a, b):
    return lax.dot_general(a.astype(jnp.bfloat16), b.astype(jnp.bfloat16),
                           (((1,), (1,)), ((), ())),
                           preferred_element_type=F32)


def _mmTT(a, b):
    return lax.dot_general(a, b, (((0,), (0,)), ((), ())),
                           preferred_element_type=F32,
                           precision=lax.Precision.HIGHEST)


def _fps_body(xs_ref, out_ref):
    x0 = xs_ref[0]
    x1 = xs_ref[1]
    x2 = xs_ref[2]
    pidx = (lax.broadcasted_iota(jnp.int32, (8, 512), 0) * 512
            + lax.broadcasted_iota(jnp.int32, (8, 512), 1))
    rsel = lax.broadcasted_iota(jnp.int32, (8, 1024), 0)
    csel = lax.broadcasted_iota(jnp.int32, (8, 1024), 1)

    def _amax2(v):
        return jnp.max(jnp.max(v, axis=1, keepdims=True), axis=0, keepdims=True)

    def _amin2(v):
        return jnp.min(jnp.min(v, axis=1, keepdims=True), axis=0, keepdims=True)

    def body(i, st):
        dist, far, ps = st
        fm = pidx == far
        c0 = _amax2(jnp.where(fm, x0, NEG))
        c1 = _amax2(jnp.where(fm, x1, NEG))
        c2 = _amax2(jnp.where(fm, x2, NEG))
        d = (x0 - c0) ** 2 + (x1 - c1) ** 2 + (x2 - c2) ** 2
        dist = jnp.minimum(dist, d)
        m = _amax2(dist)
        far = _amin2(jnp.where(dist == m, pidx, jnp.int32(1 << 30)))
        cb = jnp.where(rsel == 0, c0, jnp.where(rsel == 1, c1,
                       jnp.where(rsel == 2, c2, 0.0)))
        ps = jnp.where(csel == i, cb, ps)
        return dist, far, ps

    dist0 = jnp.full((8, 512), 1e10, F32)
    ps0 = jnp.zeros((8, 1024), F32)
    far0 = jnp.zeros((1, 1), jnp.int32)
    _, _, ps = lax.fori_loop(0, 1024, body, (dist0, far0, ps0))
    out_ref[...] = ps


def _fps(xs3):
    return pl.pallas_call(
        _fps_body,
        out_shape=jax.ShapeDtypeStruct((8, 1024), F32),
    )(xs3)


def _knn_patch_body(q_ref, ref_ref, out_ref, *, k):
    q = q_ref[...]
    r = ref_ref[...]
    n = r.shape[1]
    rn = jnp.sum(r * r, axis=0, keepdims=True)
    inner = _mmTTb(q, r)
    io0 = lax.broadcasted_iota(jnp.int32, (128, 128), 0)
    io1 = lax.broadcasted_iota(jnp.int32, (128, 128), 1)
    eyef = (io0 == io1).astype(F32)
    qn_row = jnp.sum(q * q, axis=0, keepdims=True)
    qn_col = _mmT(eyef, qn_row)
    D = (qn_col - 2.0 * inner) + rn
    iol = lax.broadcasted_iota(jnp.int32, (128, n), 1)
    for s in range(k):
        m = jnp.min(D, axis=1, keepdims=True)
        idx = jnp.min(jnp.where(D == m, iol, jnp.int32(1 << 30)),
                      axis=1, keepdims=True)
        oh = iol == idx
        if s > 0:
            patch = _mmT(r, oh.astype(F32)) - q
            out_ref[s - 1] = patch
        D = jnp.where(oh, BIG, D)


def _knn_patch(ps8, ref8, k):
    n = ref8.shape[1]
    return pl.pallas_call(
        functools.partial(_knn_patch_body, k=k),
        grid=(8,),
        in_specs=[
            pl.BlockSpec((8, 128), lambda b: (0, b)),
            pl.BlockSpec((8, n), lambda b: (0, 0)),
        ],
        out_specs=pl.BlockSpec((k - 1, 8, 128), lambda b: (0, 0, b)),
        out_shape=jax.ShapeDtypeStruct((k - 1, 8, 1024), F32),
    )(ps8, ref8)


def _seg_max4(a):
    n = a.shape[1] // 4
    return jnp.maximum(
        jnp.maximum(a[:, :n], a[:, n:2 * n]),
        jnp.maximum(a[:, 2 * n:3 * n], a[:, 3 * n:]))


def _loc_body(x_ref, w1, w2, w3, w4, w5, out_ref):
    x0 = x_ref[...]
    a = _mmb(w1[:, 0:3], x0[0:3, :])
    bv = _mmb(w1[:, 3:6], x0[0:3, :])
    t = _seg_max4(a)
    x1 = jax.nn.relu(jnp.concatenate([t, t, t, t], axis=1) + bv)
    a = _mmb(w2[:, 0:64], x1)
    bv = _mmb(w2[:, 64:128], x1)
    t = _seg_max4(a)
    x2 = jax.nn.relu(jnp.concatenate([t, t, t, t], axis=1) + bv)
    a = _mmb(w3[:, 0:64], x2)
    bv = _mmb(w3[:, 64:128], x2)
    t = _seg_max4(a)
    x3 = jax.nn.relu(jnp.concatenate([t, t, t, t], axis=1) + bv)
    a = _mmb(w4[:, 0:128], x3)
    bv = _mmb(w4[:, 128:256], x3)
    t = _seg_max4(a)
    x4 = jax.nn.relu(jnp.concatenate([t, t, t, t], axis=1) + bv)
    cat = jnp.concatenate([x1, x2, x3, x4], axis=0)
    h = _mmb(w5[...], cat)
    out_ref[...] = jnp.tanh(_seg_max4(h))


def _loc_dgcnn(x, ws):
    return pl.pallas_call(
        _loc_body,
        out_shape=jax.ShapeDtypeStruct((512, 1024), F32),
    )(x, *ws)


def _se_fused_body(x_ref, w1, w2, w3, w4, w5, out_ref):
    io0 = lax.broadcasted_iota(jnp.int32, (128, 128), 0)
    io1 = lax.broadcasted_iota(jnp.int32, (128, 128), 1)
    eyef = (io0 == io1).astype(F32)
    grpf = ((io0 // 16) == (io1 // 16)).astype(F32)
    r16 = ((io0 % 16) == io1)[:, 0:16].astype(F32)
    iot = lax.broadcasted_iota(jnp.int32, (16, 128), 0)
    sl8 = (lax.broadcasted_iota(jnp.int32, (8, 128), 0) * 16
           == lax.broadcasted_iota(jnp.int32, (8, 128), 1)).astype(F32)

    def layer(x, w, cin):
        a = _mmb(w[:, 0:cin], x[0:cin, :])
        bv = _mmb(w[:, cin:2 * cin], x[0:cin, :])
        g = _mmTTb(x, x)
        n_row = jnp.sum(x * x, axis=0, keepdims=True)
        n_col = _mmT(eyef, n_row)
        z = grpf * (-n_col - (-2.0 * g))
        pd3 = _mmTT(r16, z) - n_row
        macc = None
        for s in range(4):
            mcol = jnp.max(pd3, axis=0, keepdims=True)
            tidx = jnp.min(jnp.where(pd3 == mcol, iot, jnp.int32(99)),
                           axis=0, keepdims=True)
            oh3 = iot == tidx
            ohf = jnp.concatenate([oh3.astype(F32)] * 8, axis=0) * grpf
            ms = _mm(a, ohf)
            macc = ms if s == 0 else jnp.maximum(macc, ms)
            pd3 = jnp.where(oh3, NEG, pd3)
        return jax.nn.relu(macc + bv)

    for sb in range(16):
        x = x_ref[:, sb * 128:(sb + 1) * 128]
        x1 = layer(x, w1, 3)
        x2 = layer(x1, w2, 64)
        x3 = layer(x2, w3, 64)
        x4 = layer(x3, w4, 128)
        cat = jnp.concatenate([x1, x2, x3, x4], axis=0)
        h5 = _mmb(w5[...], cat)
        m = h5
        for sh in (1, 2, 4, 8):
            m = jnp.maximum(m, pltpu.roll(m, 128 - sh, 1))
        se_t = jnp.tanh(_mmT(sl8, m))
        out_ref[sb * 8:(sb + 1) * 8, :] = se_t


def _se_fused(xse, ws):
    return pl.pallas_call(
        _se_fused_body,
        grid=(8,),
        in_specs=[pl.BlockSpec((8, 2048), lambda b: (0, b))]
        + [pl.BlockSpec(w.shape, lambda b: (0, 0)) for w in ws],
        out_specs=pl.BlockSpec((128, 512), lambda b: (b, 0)),
        out_shape=jax.ShapeDtypeStruct((1024, 512), F32),
    )(xse, *ws)


def _head_body(se_ref, loc_ref, m1w_ref, m1b_ref, m2w_ref, m2b_ref,
               rw_ref, rb_ref, out_ref):
    se_t = se_ref[...]
    loc = loc_ref[...]
    h = jax.nn.relu(_mmTb(m1w_ref[:, 0:512], se_t)
                    + _mmb(m1w_ref[:, 512:1024], loc) + m1b_ref[...])
    h = jax.nn.relu(_mmb(m2w_ref[...], h) + m2b_ref[...])
    f = _mmb(rw_ref[...], h) + rb_ref[...]
    out_ref[...] = jnp.concatenate([f, jnp.zeros((5, 1024), F32)], axis=0)


def _head(se_t, loc, m1w, m1b, m2w, m2b, rw, rb):
    return pl.pallas_call(
        _head_body,
        out_shape=jax.ShapeDtypeStruct((8, 1024), F32),
    )(se_t, loc, m1w, m1b, m2w, m2b, rw, rb)


def kernel(xyz, A1, A2, A3, A4, A5, B1, B2, B3, B4, B5,
           M1w, M1b, M2w, M2b, Rw, Rb):
    xs = xyz[0]
    xs3 = xs.reshape(3, 8, 512)
    xs8 = jnp.concatenate([xs, jnp.zeros((5, 4096), F32)], axis=0)

    ps8 = _fps(xs3)

    p1 = _knn_patch(ps8, xs8, 5)
    xloc = p1.transpose(1, 0, 2).reshape(8, 4096)
    loc = _loc_dgcnn(xloc, (B1, B2, B3, B4, B5))

    p2 = _knn_patch(ps8, ps8, 17)
    xse = p2.transpose(1, 2, 0).reshape(8, 16384)
    se_t = _se_fused(xse, (A1, A2, A3, A4, A5))

    feat8 = _head(se_t, loc, M1w, M1b.reshape(512, 1),
                  M2w, M2b.reshape(256, 1), Rw, Rb.reshape(3, 1))
    ps_out = ps8[0:3][None]
    feat = feat8[0:3][None]
    return (ps_out, feat)
